# speedup vs baseline: 6.3271x; 6.3271x over previous
import functools

import jax
import jax.numpy as jnp
from jax import lax
from jax.experimental import pallas as pl
from jax.experimental.pallas import tpu as pltpu
from jax.experimental.pallas import tpu_sc as plsc

B, S, D = 4096, 200, 128
NC, NS = 2, 16
NW = NC * NS
ROWS_PER_W = (B * S) // NW
CHUNK = 128
N_CHUNK = ROWS_PER_W // CHUNK


def _emb_body(x_hbm, w_hbm, out_hbm, idx_v, rows_v, sem):
    c = lax.axis_index("c")
    s = lax.axis_index("s")
    wid = s * NC + c

    pltpu.sync_copy(x_hbm.at[wid], idx_v)

    def body(j, carry):
        pltpu.async_copy(w_hbm.at[idx_v.at[j]], rows_v, sem).wait()
        pltpu.sync_copy(rows_v, out_hbm.at[wid, j])
        return carry

    lax.fori_loop(0, N_CHUNK, body, 0)


@jax.jit
def kernel(x, W):
    xf = x.reshape(NW, N_CHUNK, CHUNK).astype(jnp.int32)
    mesh = plsc.VectorSubcoreMesh(core_axis_name="c", subcore_axis_name="s")
    f = pl.kernel(
        _emb_body,
        out_type=jax.ShapeDtypeStruct((NW, N_CHUNK, CHUNK, D), jnp.float32),
        mesh=mesh,
        scratch_types=[
            pltpu.VMEM((N_CHUNK, CHUNK), jnp.int32),
            pltpu.VMEM((CHUNK, D), jnp.float32),
            pltpu.SemaphoreType.DMA,
        ],
    )
    out = f(xf, W)
    return out.reshape(B, S, D)


# speedup vs baseline: 9.0991x; 1.4381x over previous
import functools

import jax
import jax.numpy as jnp
from jax import lax
from jax.experimental import pallas as pl
from jax.experimental.pallas import tpu as pltpu
from jax.experimental.pallas import tpu_sc as plsc

B, S, D = 4096, 200, 128
NC, NS = 2, 16
NW = NC * NS
ROWS_PER_W = (B * S) // NW
CHUNK = 128
N_CHUNK = ROWS_PER_W // CHUNK
NBUF = 5
N_GROUP = N_CHUNK // NBUF


def _emb_body(x_hbm, w_hbm, out_hbm, idx_v, *scratch):
    bufs = scratch[:NBUF]
    gsems = scratch[NBUF:2 * NBUF]
    wsems = scratch[2 * NBUF:3 * NBUF]

    c = lax.axis_index("c")
    s = lax.axis_index("s")
    wid = s * NC + c

    pltpu.sync_copy(x_hbm.at[wid], idx_v)

    for b in range(NBUF):
        pltpu.async_copy(w_hbm.at[idx_v.at[b]], bufs[b], gsems[b])

    def body(g, carry):
        for b in range(NBUF):
            j = g * NBUF + b
            pltpu.make_async_copy(
                w_hbm.at[idx_v.at[j]], bufs[b], gsems[b]).wait()
            pltpu.async_copy(bufs[b], out_hbm.at[wid, j], wsems[b])
        for b in range(NBUF):
            jn = (g + 1) * NBUF + b
            pltpu.make_async_copy(
                bufs[b], out_hbm.at[wid, 0], wsems[b]).wait()

            @pl.when(jn < N_CHUNK)
            def _():
                pltpu.async_copy(w_hbm.at[idx_v.at[jn]], bufs[b], gsems[b])

        return carry

    lax.fori_loop(0, N_GROUP, body, 0)


@jax.jit
def kernel(x, W):
    xf = x.reshape(NW, N_CHUNK, CHUNK).astype(jnp.int32)
    mesh = plsc.VectorSubcoreMesh(core_axis_name="c", subcore_axis_name="s")
    scratch = (
        [pltpu.VMEM((CHUNK, D), jnp.float32) for _ in range(NBUF)]
        + [pltpu.SemaphoreType.DMA for _ in range(2 * NBUF)]
    )
    f = pl.kernel(
        _emb_body,
        out_type=jax.ShapeDtypeStruct((NW, N_CHUNK, CHUNK, D), jnp.float32),
        mesh=mesh,
        scratch_types=[pltpu.VMEM((N_CHUNK, CHUNK), jnp.int32)] + scratch,
    )
    out = f(xf, W)
    return out.reshape(B, S, D)
